# baseline (speedup 1.0000x reference)
import functools

import jax
import jax.numpy as jnp
from jax import lax
from jax.experimental import pallas as pl
from jax.experimental.pallas import tpu as pltpu
from jax.experimental.pallas import tpu_sc as plsc

N = 10000
NP = 10240
E = 320000
H = 128
C = 64
NC = 2
NS = 16
NW = NC * NS
CH = 128
NCHUNK = 80
EP = NW * NCHUNK * CH
ROWS_PER_TILE = NP // NS
NBUF = 2
NBUF_D = 4

BM = 256
GRID = NP // BM

_mesh = plsc.VectorSubcoreMesh(core_axis_name="c", subcore_axis_name="s")


def _unpack(pidx, sidx, didx, j, b):
    for k in range(CH // 16):
        v = pidx[j, pl.ds(k * 16, 16)]
        if sidx is not None:
            sidx[b, pl.ds(k * 16, 16)] = lax.bitwise_and(v, 0xFFFF)
        didx[b, pl.ds(k * 16, 16)] = lax.shift_right_logical(v, 16)


@functools.partial(
    pl.kernel,
    out_type=jax.ShapeDtypeStruct((NC, NP, H), jnp.float32),
    mesh=_mesh,
    scratch_types=[
        pltpu.VMEM((NCHUNK, CH), jnp.int32),
        pltpu.VMEM((NBUF_D, CH), jnp.int32),
        pltpu.VMEM((CH, H), jnp.float32),
        pltpu.VMEM_SHARED((NP, H), jnp.float32),
    ]
    + [pltpu.SemaphoreType.DMA] * NBUF_D,
)
def _deg_kernel(pidx3, ones_hbm, zeros_hbm, degp, pidx, didx, ones_v, acc, *ssem):
    cid = lax.axis_index("c")
    sid = lax.axis_index("s")
    wid = cid * NS + sid
    sl = pl.ds(sid * ROWS_PER_TILE, ROWS_PER_TILE)
    pltpu.sync_copy(zeros_hbm, acc.at[sl])
    pltpu.sync_copy(pidx3.at[wid], pidx)
    pltpu.sync_copy(ones_hbm, ones_v)
    plsc.subcore_barrier()

    for b in range(NBUF_D):
        _unpack(pidx, None, didx, b, b)
        pltpu.async_copy(ones_v, acc.at[didx.at[b]], ssem[b], add=True)

    @pl.loop(1, NCHUNK // NBUF_D)
    def _(r):
        jo = r * NBUF_D
        for b in range(NBUF_D):
            pltpu.make_async_copy(ones_v, acc.at[didx.at[b]], ssem[b]).wait()
            _unpack(pidx, None, didx, jo + b, b)
            pltpu.async_copy(ones_v, acc.at[didx.at[b]], ssem[b], add=True)

    for b in range(NBUF_D):
        pltpu.make_async_copy(ones_v, acc.at[didx.at[b]], ssem[b]).wait()

    plsc.subcore_barrier()
    pltpu.sync_copy(acc.at[sl], degp.at[cid].at[sl])


CA = 40
CB = 120
MAXC = max(CA, CB)
TOTC = NS * (CA + CB)


@functools.partial(
    pl.kernel,
    out_type=jax.ShapeDtypeStruct((NC, NP, H), jnp.float32),
    mesh=_mesh,
    scratch_types=[
        pltpu.VMEM((MAXC, CH), jnp.int32),
        pltpu.VMEM((NBUF, CH), jnp.int32),
        pltpu.VMEM((NBUF, CH), jnp.int32),
        pltpu.VMEM((NBUF, CH, H), jnp.float32),
        pltpu.VMEM_SHARED((NP, H), jnp.float32),
    ]
    + [pltpu.SemaphoreType.DMA] * (2 * NBUF),
)
def _agg_kernel(g_hbm, pidxf, zeros_hbm, part, pidx, sidx, didx, rows, acc, *sems):
    gsem = sems[:NBUF]
    ssem = sems[NBUF:]
    cid = lax.axis_index("c")
    sid = lax.axis_index("s")
    cnt = lax.select(cid == 0, CA, CB)
    base = lax.select(cid == 0, sid * CA, NS * CA + sid * CB)
    sl = pl.ds(sid * ROWS_PER_TILE, ROWS_PER_TILE)
    pltpu.sync_copy(zeros_hbm, acc.at[sl])
    pltpu.sync_copy(pidxf.at[pl.ds(base, MAXC)], pidx)
    plsc.subcore_barrier()

    for b in range(NBUF):
        _unpack(pidx, sidx, didx, b, b)
        pltpu.async_copy(g_hbm.at[sidx.at[b]], rows.at[b], gsem[b])

    @pl.loop(0, lax.div(cnt, NBUF))
    def _(r):
        jo = r * NBUF
        for b in range(NBUF):
            pltpu.make_async_copy(g_hbm.at[sidx.at[b]], rows.at[b], gsem[b]).wait()
            pltpu.async_copy(rows.at[b], acc.at[didx.at[b]], ssem[b], add=True)
        for b in range(NBUF):
            j = jo + b
            pltpu.make_async_copy(rows.at[b], acc.at[didx.at[b]], ssem[b]).wait()

            @pl.when(j + NBUF < cnt)
            def _():
                _unpack(pidx, sidx, didx, j + NBUF, b)
                pltpu.async_copy(g_hbm.at[sidx.at[b]], rows.at[b], gsem[b])

    plsc.subcore_barrier()
    pltpu.sync_copy(acc.at[sl], part.at[cid].at[sl])


def _enc_body(x_ref, w_ref, b_ref, degp_ref, h_ref, g_ref, dinv_ref):
    h = jnp.dot(x_ref[...], w_ref[...], preferred_element_type=jnp.float32)
    h = h + b_ref[...]
    h = jnp.where(h >= 0.0, h, 0.01 * h)
    deg = degp_ref[0, :, 0:1] + degp_ref[1, :, 0:1] + 1.0
    dinv = lax.rsqrt(deg)
    h_ref[...] = h
    g_ref[...] = h * dinv
    dinv_ref[...] = jnp.broadcast_to(dinv, dinv_ref.shape)


def _ab_from_lam(lam_ref):
    lam = 1.0 + jnp.maximum(lam_ref[0, 0], 0.0)
    return (2.0 * lam - 2.0) / lam, 2.0 / lam


def _comb_body(h_ref, p_ref, dinv_ref, lam_ref, hn_ref, gn_ref):
    a1, a2 = _ab_from_lam(lam_ref)
    dinv = dinv_ref[:, 0:1]
    h = h_ref[...]
    agg = p_ref[0] + p_ref[1]
    hn = a1 * h + a2 * (dinv * agg + (dinv * dinv) * h)
    hn_ref[...] = hn
    gn_ref[...] = dinv * hn


def _final_body(h1_ref, h2_ref, p_ref, dinv_ref, lam_ref, wf_ref, bf_ref, o_ref):
    a1, a2 = _ab_from_lam(lam_ref)
    dinv = dinv_ref[:, 0:1]
    h2 = h2_ref[...]
    agg = p_ref[0] + p_ref[1]
    h3 = a1 * h2 + a2 * (dinv * agg + (dinv * dinv) * h2)
    o = jnp.dot(h1_ref[...], wf_ref[0:H], preferred_element_type=jnp.float32)
    o += jnp.dot(h2, wf_ref[H:2 * H], preferred_element_type=jnp.float32)
    o += jnp.dot(h3, wf_ref[2 * H:3 * H], preferred_element_type=jnp.float32)
    o_ref[...] = o + bf_ref[...]


def _row_spec(w):
    return pl.BlockSpec((BM, w), lambda i: (i, 0))


def _part_spec(w):
    return pl.BlockSpec((NC, BM, w), lambda i: (0, i, 0))


_full = lambda shape: pl.BlockSpec(shape, lambda i: tuple(0 for _ in shape))

_enc_call = pl.pallas_call(
    _enc_body,
    grid=(GRID,),
    in_specs=[_row_spec(H), _full((H, H)), _full((1, H)), _part_spec(H)],
    out_specs=[_row_spec(H), _row_spec(H), _row_spec(16)],
    out_shape=[jax.ShapeDtypeStruct((NP, H), jnp.float32)] * 2
    + [jax.ShapeDtypeStruct((NP, 16), jnp.float32)],
)

_comb_call = pl.pallas_call(
    _comb_body,
    grid=(GRID,),
    in_specs=[_row_spec(H), _part_spec(H), _row_spec(16), _full((1, 1))],
    out_specs=[_row_spec(H), _row_spec(H)],
    out_shape=[jax.ShapeDtypeStruct((NP, H), jnp.float32)] * 2,
)

_final_call = pl.pallas_call(
    _final_body,
    grid=(GRID,),
    in_specs=[_row_spec(H), _row_spec(H), _part_spec(H), _row_spec(16),
              _full((1, 1)), _full((3 * H, C)), _full((1, C))],
    out_specs=_row_spec(C),
    out_shape=jax.ShapeDtypeStruct((NP, C), jnp.float32),
)


@jax.jit
def kernel(x, edge_index, W_feat, b_feat, lambdas, W_final, b_final):
    pad = jnp.full((EP - E,), N, jnp.int32)
    srcp = jnp.concatenate([edge_index[0], pad])
    dstp = jnp.concatenate([edge_index[1], pad])
    packed = (dstp << 16) | srcp
    pidx3 = packed.reshape(NW, NCHUNK, CH)
    dummy = jnp.full((MAXC, CH), (N << 16) | N, jnp.int32)
    pidxf = jnp.concatenate([packed.reshape(TOTC, CH), dummy])
    x_pad = jnp.zeros((NP, H), jnp.float32).at[:N].set(x)

    ones_d = jnp.ones((CH, H), jnp.float32)
    zeros_a = jnp.zeros((ROWS_PER_TILE, H), jnp.float32)

    degp = _deg_kernel(pidx3, ones_d, zeros_a)
    h0, g, dinv = _enc_call(x_pad, W_feat, b_feat.reshape(1, H), degp)

    p1 = _agg_kernel(g, pidxf, zeros_a)
    h1, g = _comb_call(h0, p1, dinv, lambdas[0].reshape(1, 1))

    p2 = _agg_kernel(g, pidxf, zeros_a)
    h2, g = _comb_call(h1, p2, dinv, lambdas[1].reshape(1, 1))

    p3 = _agg_kernel(g, pidxf, zeros_a)
    out = _final_call(h1, h2, p3, dinv, lambdas[2].reshape(1, 1),
                      W_final, b_final.reshape(1, C))
    return out[:N]


# speedup vs baseline: 1.1369x; 1.1369x over previous
import functools

import jax
import jax.numpy as jnp
from jax import lax
from jax.experimental import pallas as pl
from jax.experimental.pallas import tpu as pltpu
from jax.experimental.pallas import tpu_sc as plsc

N = 10000
NP = 10240
E = 320000
H = 128
C = 64
NC = 2
NS = 16
NW = NC * NS
CH = 128
NCHUNK = 80
EP = NW * NCHUNK * CH
ROWS_PER_TILE = NP // NS
NBUF = 2
NBUF_D = 4

BM = 256
GRID = NP // BM

_mesh = plsc.VectorSubcoreMesh(core_axis_name="c", subcore_axis_name="s")


def _unpack(pidx, sidx, didx, j, b):
    for k in range(CH // 16):
        v = pidx[j, pl.ds(k * 16, 16)]
        if sidx is not None:
            sidx[b, pl.ds(k * 16, 16)] = lax.bitwise_and(v, 0xFFFF)
        didx[b, pl.ds(k * 16, 16)] = lax.shift_right_logical(v, 16)


@functools.partial(
    pl.kernel,
    out_type=jax.ShapeDtypeStruct((NC, NP, H), jnp.float32),
    mesh=_mesh,
    scratch_types=[
        pltpu.VMEM((NCHUNK, CH), jnp.int32),
        pltpu.VMEM((NBUF_D, CH), jnp.int32),
        pltpu.VMEM((CH, H), jnp.float32),
        pltpu.VMEM_SHARED((NP, H), jnp.float32),
    ]
    + [pltpu.SemaphoreType.DMA] * NBUF_D,
)
def _deg_kernel(pidx3, ones_hbm, zeros_hbm, degp, pidx, didx, ones_v, acc, *ssem):
    cid = lax.axis_index("c")
    sid = lax.axis_index("s")
    wid = cid * NS + sid
    sl = pl.ds(sid * ROWS_PER_TILE, ROWS_PER_TILE)
    pltpu.sync_copy(zeros_hbm, acc.at[sl])
    pltpu.sync_copy(pidx3.at[wid], pidx)
    pltpu.sync_copy(ones_hbm, ones_v)
    plsc.subcore_barrier()

    for b in range(NBUF_D):
        _unpack(pidx, None, didx, b, b)
        pltpu.async_copy(ones_v, acc.at[didx.at[b]], ssem[b], add=True)

    @pl.loop(1, NCHUNK // NBUF_D)
    def _(r):
        jo = r * NBUF_D
        for b in range(NBUF_D):
            pltpu.make_async_copy(ones_v, acc.at[didx.at[b]], ssem[b]).wait()
            _unpack(pidx, None, didx, jo + b, b)
            pltpu.async_copy(ones_v, acc.at[didx.at[b]], ssem[b], add=True)

    for b in range(NBUF_D):
        pltpu.make_async_copy(ones_v, acc.at[didx.at[b]], ssem[b]).wait()

    plsc.subcore_barrier()
    pltpu.sync_copy(acc.at[sl], degp.at[cid].at[sl])


CA = 120
CB = 40
MAXC = max(CA, CB)
TOTC = NS * (CA + CB)


@functools.partial(
    pl.kernel,
    out_type=jax.ShapeDtypeStruct((NC, NP, H), jnp.float32),
    mesh=_mesh,
    scratch_types=[
        pltpu.VMEM((MAXC, CH), jnp.int32),
        pltpu.VMEM((NBUF, CH), jnp.int32),
        pltpu.VMEM((NBUF, CH), jnp.int32),
        pltpu.VMEM((NBUF, CH, H), jnp.float32),
        pltpu.VMEM_SHARED((NP, H), jnp.float32),
    ]
    + [pltpu.SemaphoreType.DMA] * (2 * NBUF),
)
def _agg_kernel(g_hbm, pidxf, zeros_hbm, part, pidx, sidx, didx, rows, acc, *sems):
    gsem = sems[:NBUF]
    ssem = sems[NBUF:]
    cid = lax.axis_index("c")
    sid = lax.axis_index("s")
    cnt = lax.select(cid == 0, CA, CB)
    base = lax.select(cid == 0, sid * CA, NS * CA + sid * CB)
    sl = pl.ds(sid * ROWS_PER_TILE, ROWS_PER_TILE)
    pltpu.sync_copy(zeros_hbm, acc.at[sl])
    pltpu.sync_copy(pidxf.at[pl.ds(base, MAXC)], pidx)
    plsc.subcore_barrier()

    for b in range(NBUF):
        _unpack(pidx, sidx, didx, b, b)
        pltpu.async_copy(g_hbm.at[sidx.at[b]], rows.at[b], gsem[b])

    @pl.loop(0, lax.div(cnt, NBUF))
    def _(r):
        jo = r * NBUF
        for b in range(NBUF):
            pltpu.make_async_copy(g_hbm.at[sidx.at[b]], rows.at[b], gsem[b]).wait()
            pltpu.async_copy(rows.at[b], acc.at[didx.at[b]], ssem[b], add=True)
        for b in range(NBUF):
            j = jo + b
            pltpu.make_async_copy(rows.at[b], acc.at[didx.at[b]], ssem[b]).wait()

            @pl.when(j + NBUF < cnt)
            def _():
                _unpack(pidx, sidx, didx, j + NBUF, b)
                pltpu.async_copy(g_hbm.at[sidx.at[b]], rows.at[b], gsem[b])

    plsc.subcore_barrier()
    pltpu.sync_copy(acc.at[sl], part.at[cid].at[sl])


def _enc_body(x_ref, w_ref, b_ref, degp_ref, h_ref, g_ref, dinv_ref):
    h = jnp.dot(x_ref[...], w_ref[...], preferred_element_type=jnp.float32)
    h = h + b_ref[...]
    h = jnp.where(h >= 0.0, h, 0.01 * h)
    deg = degp_ref[0, :, 0:1] + degp_ref[1, :, 0:1] + 1.0
    dinv = lax.rsqrt(deg)
    h_ref[...] = h
    g_ref[...] = h * dinv
    dinv_ref[...] = jnp.broadcast_to(dinv, dinv_ref.shape)


def _ab_from_lam(lam_ref):
    lam = 1.0 + jnp.maximum(lam_ref[0, 0], 0.0)
    return (2.0 * lam - 2.0) / lam, 2.0 / lam


def _comb_body(h_ref, p_ref, dinv_ref, lam_ref, hn_ref, gn_ref):
    a1, a2 = _ab_from_lam(lam_ref)
    dinv = dinv_ref[:, 0:1]
    h = h_ref[...]
    agg = p_ref[0] + p_ref[1]
    hn = a1 * h + a2 * (dinv * agg + (dinv * dinv) * h)
    hn_ref[...] = hn
    gn_ref[...] = dinv * hn


def _final_body(h1_ref, h2_ref, p_ref, dinv_ref, lam_ref, wf_ref, bf_ref, o_ref):
    a1, a2 = _ab_from_lam(lam_ref)
    dinv = dinv_ref[:, 0:1]
    h2 = h2_ref[...]
    agg = p_ref[0] + p_ref[1]
    h3 = a1 * h2 + a2 * (dinv * agg + (dinv * dinv) * h2)
    o = jnp.dot(h1_ref[...], wf_ref[0:H], preferred_element_type=jnp.float32)
    o += jnp.dot(h2, wf_ref[H:2 * H], preferred_element_type=jnp.float32)
    o += jnp.dot(h3, wf_ref[2 * H:3 * H], preferred_element_type=jnp.float32)
    o_ref[...] = o + bf_ref[...]


def _row_spec(w):
    return pl.BlockSpec((BM, w), lambda i: (i, 0))


def _part_spec(w):
    return pl.BlockSpec((NC, BM, w), lambda i: (0, i, 0))


_full = lambda shape: pl.BlockSpec(shape, lambda i: tuple(0 for _ in shape))

_enc_call = pl.pallas_call(
    _enc_body,
    grid=(GRID,),
    in_specs=[_row_spec(H), _full((H, H)), _full((1, H)), _part_spec(H)],
    out_specs=[_row_spec(H), _row_spec(H), _row_spec(16)],
    out_shape=[jax.ShapeDtypeStruct((NP, H), jnp.float32)] * 2
    + [jax.ShapeDtypeStruct((NP, 16), jnp.float32)],
)

_comb_call = pl.pallas_call(
    _comb_body,
    grid=(GRID,),
    in_specs=[_row_spec(H), _part_spec(H), _row_spec(16), _full((1, 1))],
    out_specs=[_row_spec(H), _row_spec(H)],
    out_shape=[jax.ShapeDtypeStruct((NP, H), jnp.float32)] * 2,
)

_final_call = pl.pallas_call(
    _final_body,
    grid=(GRID,),
    in_specs=[_row_spec(H), _row_spec(H), _part_spec(H), _row_spec(16),
              _full((1, 1)), _full((3 * H, C)), _full((1, C))],
    out_specs=_row_spec(C),
    out_shape=jax.ShapeDtypeStruct((NP, C), jnp.float32),
)


@jax.jit
def kernel(x, edge_index, W_feat, b_feat, lambdas, W_final, b_final):
    pad = jnp.full((EP - E,), N, jnp.int32)
    srcp = jnp.concatenate([edge_index[0], pad])
    dstp = jnp.concatenate([edge_index[1], pad])
    packed = (dstp << 16) | srcp
    pidx3 = packed.reshape(NW, NCHUNK, CH)
    dummy = jnp.full((MAXC, CH), (N << 16) | N, jnp.int32)
    pidxf = jnp.concatenate([packed.reshape(TOTC, CH), dummy])
    x_pad = jnp.zeros((NP, H), jnp.float32).at[:N].set(x)

    ones_d = jnp.ones((CH, H), jnp.float32)
    zeros_a = jnp.zeros((ROWS_PER_TILE, H), jnp.float32)

    degp = _deg_kernel(pidx3, ones_d, zeros_a)
    h0, g, dinv = _enc_call(x_pad, W_feat, b_feat.reshape(1, H), degp)

    p1 = _agg_kernel(g, pidxf, zeros_a)
    h1, g = _comb_call(h0, p1, dinv, lambdas[0].reshape(1, 1))

    p2 = _agg_kernel(g, pidxf, zeros_a)
    h2, g = _comb_call(h1, p2, dinv, lambdas[1].reshape(1, 1))

    p3 = _agg_kernel(g, pidxf, zeros_a)
    out = _final_call(h1, h2, p3, dinv, lambdas[2].reshape(1, 1),
                      W_final, b_final.reshape(1, C))
    return out[:N]


# speedup vs baseline: 1.1423x; 1.0047x over previous
import functools

import jax
import jax.numpy as jnp
from jax import lax
from jax.experimental import pallas as pl
from jax.experimental.pallas import tpu as pltpu
from jax.experimental.pallas import tpu_sc as plsc

N = 10000
NP = 10240
E = 320000
H = 128
C = 64
NC = 2
NS = 16
NW = NC * NS
CH = 128
NCHUNK = 80
EP = NW * NCHUNK * CH
ROWS_PER_TILE = NP // NS
NBUF = 2
NBUF_D = 4

BM = 256
GRID = NP // BM

_mesh = plsc.VectorSubcoreMesh(core_axis_name="c", subcore_axis_name="s")


def _unpack(pidx, sidx, didx, j, b):
    for k in range(CH // 16):
        v = pidx[j, pl.ds(k * 16, 16)]
        if sidx is not None:
            sidx[b, pl.ds(k * 16, 16)] = lax.bitwise_and(v, 0xFFFF)
        didx[b, pl.ds(k * 16, 16)] = lax.shift_right_logical(v, 16)


@functools.partial(
    pl.kernel,
    out_type=jax.ShapeDtypeStruct((NC, NP, H), jnp.float32),
    mesh=_mesh,
    scratch_types=[
        pltpu.VMEM((NCHUNK, CH), jnp.int32),
        pltpu.VMEM((NBUF_D, CH), jnp.int32),
        pltpu.VMEM((CH, H), jnp.float32),
        pltpu.VMEM_SHARED((NP, H), jnp.float32),
    ]
    + [pltpu.SemaphoreType.DMA] * NBUF_D,
)
def _deg_kernel(pidx3, ones_hbm, zeros_hbm, degp, pidx, didx, ones_v, acc, *ssem):
    cid = lax.axis_index("c")
    sid = lax.axis_index("s")
    wid = cid * NS + sid
    sl = pl.ds(sid * ROWS_PER_TILE, ROWS_PER_TILE)
    pltpu.sync_copy(zeros_hbm, acc.at[sl])
    pltpu.sync_copy(pidx3.at[wid], pidx)
    pltpu.sync_copy(ones_hbm, ones_v)
    plsc.subcore_barrier()

    for b in range(NBUF_D):
        _unpack(pidx, None, didx, b, b)
        pltpu.async_copy(ones_v, acc.at[didx.at[b]], ssem[b], add=True)

    @pl.loop(1, NCHUNK // NBUF_D)
    def _(r):
        jo = r * NBUF_D
        for b in range(NBUF_D):
            pltpu.make_async_copy(ones_v, acc.at[didx.at[b]], ssem[b]).wait()
            _unpack(pidx, None, didx, jo + b, b)
            pltpu.async_copy(ones_v, acc.at[didx.at[b]], ssem[b], add=True)

    for b in range(NBUF_D):
        pltpu.make_async_copy(ones_v, acc.at[didx.at[b]], ssem[b]).wait()

    plsc.subcore_barrier()
    pltpu.sync_copy(acc.at[sl], degp.at[cid].at[sl])


CA = 128
CB = 32
MAXC = max(CA, CB)
TOTC = NS * (CA + CB)
NA = 10112
RPT_A = NA // NS


@functools.partial(
    pl.kernel,
    out_type=jax.ShapeDtypeStruct((NC, NP, H), jnp.float32),
    mesh=_mesh,
    scratch_types=[
        pltpu.VMEM((MAXC, CH), jnp.int32),
        pltpu.VMEM((NBUF, CH), jnp.int32),
        pltpu.VMEM((NBUF, CH), jnp.int32),
        pltpu.VMEM((NBUF, CH, H), jnp.float32),
        pltpu.VMEM_SHARED((NA, H), jnp.float32),
    ]
    + [pltpu.SemaphoreType.DMA] * (2 * NBUF),
)
def _agg_kernel(g_hbm, pidxf, zeros_hbm, part, pidx, sidx, didx, rows, acc, *sems):
    gsem = sems[:NBUF]
    ssem = sems[NBUF:]
    cid = lax.axis_index("c")
    sid = lax.axis_index("s")
    cnt = lax.select(cid == 0, CA, CB)
    base = lax.select(cid == 0, sid * CA, NS * CA + sid * CB)
    sl = pl.ds(sid * RPT_A, RPT_A)
    pltpu.sync_copy(zeros_hbm.at[pl.ds(0, RPT_A)], acc.at[sl])
    pltpu.sync_copy(pidxf.at[pl.ds(base, MAXC)], pidx)
    plsc.subcore_barrier()

    for b in range(NBUF):
        _unpack(pidx, sidx, didx, b, b)
        pltpu.async_copy(g_hbm.at[sidx.at[b]], rows.at[b], gsem[b])

    @pl.loop(0, lax.div(cnt, NBUF))
    def _(r):
        jo = r * NBUF
        for b in range(NBUF):
            pltpu.make_async_copy(g_hbm.at[sidx.at[b]], rows.at[b], gsem[b]).wait()
            pltpu.async_copy(rows.at[b], acc.at[didx.at[b]], ssem[b], add=True)
        for b in range(NBUF):
            j = jo + b
            pltpu.make_async_copy(rows.at[b], acc.at[didx.at[b]], ssem[b]).wait()

            @pl.when(j + NBUF < cnt)
            def _():
                _unpack(pidx, sidx, didx, j + NBUF, b)
                pltpu.async_copy(g_hbm.at[sidx.at[b]], rows.at[b], gsem[b])

    plsc.subcore_barrier()
    pltpu.sync_copy(acc.at[sl], part.at[cid].at[sl])


def _enc_body(x_ref, w_ref, b_ref, degp_ref, h_ref, g_ref, dinv_ref):
    h = jnp.dot(x_ref[...], w_ref[...], preferred_element_type=jnp.float32)
    h = h + b_ref[...]
    h = jnp.where(h >= 0.0, h, 0.01 * h)
    deg = degp_ref[0, :, 0:1] + degp_ref[1, :, 0:1] + 1.0
    dinv = lax.rsqrt(deg)
    h_ref[...] = h
    g_ref[...] = h * dinv
    dinv_ref[...] = jnp.broadcast_to(dinv, dinv_ref.shape)


def _ab_from_lam(lam_ref):
    lam = 1.0 + jnp.maximum(lam_ref[0, 0], 0.0)
    return (2.0 * lam - 2.0) / lam, 2.0 / lam


def _comb_body(h_ref, p_ref, dinv_ref, lam_ref, hn_ref, gn_ref):
    a1, a2 = _ab_from_lam(lam_ref)
    dinv = dinv_ref[:, 0:1]
    h = h_ref[...]
    agg = p_ref[0] + p_ref[1]
    hn = a1 * h + a2 * (dinv * agg + (dinv * dinv) * h)
    hn_ref[...] = hn
    gn_ref[...] = dinv * hn


def _final_body(h1_ref, h2_ref, p_ref, dinv_ref, lam_ref, wf_ref, bf_ref, o_ref):
    a1, a2 = _ab_from_lam(lam_ref)
    dinv = dinv_ref[:, 0:1]
    h2 = h2_ref[...]
    agg = p_ref[0] + p_ref[1]
    h3 = a1 * h2 + a2 * (dinv * agg + (dinv * dinv) * h2)
    o = jnp.dot(h1_ref[...], wf_ref[0:H], preferred_element_type=jnp.float32)
    o += jnp.dot(h2, wf_ref[H:2 * H], preferred_element_type=jnp.float32)
    o += jnp.dot(h3, wf_ref[2 * H:3 * H], preferred_element_type=jnp.float32)
    o_ref[...] = o + bf_ref[...]


def _row_spec(w):
    return pl.BlockSpec((BM, w), lambda i: (i, 0))


def _part_spec(w):
    return pl.BlockSpec((NC, BM, w), lambda i: (0, i, 0))


_full = lambda shape: pl.BlockSpec(shape, lambda i: tuple(0 for _ in shape))

_enc_call = pl.pallas_call(
    _enc_body,
    grid=(GRID,),
    in_specs=[_row_spec(H), _full((H, H)), _full((1, H)), _part_spec(H)],
    out_specs=[_row_spec(H), _row_spec(H), _row_spec(16)],
    out_shape=[jax.ShapeDtypeStruct((NP, H), jnp.float32)] * 2
    + [jax.ShapeDtypeStruct((NP, 16), jnp.float32)],
)

_comb_call = pl.pallas_call(
    _comb_body,
    grid=(GRID,),
    in_specs=[_row_spec(H), _part_spec(H), _row_spec(16), _full((1, 1))],
    out_specs=[_row_spec(H), _row_spec(H)],
    out_shape=[jax.ShapeDtypeStruct((NP, H), jnp.float32)] * 2,
)

_final_call = pl.pallas_call(
    _final_body,
    grid=(GRID,),
    in_specs=[_row_spec(H), _row_spec(H), _part_spec(H), _row_spec(16),
              _full((1, 1)), _full((3 * H, C)), _full((1, C))],
    out_specs=_row_spec(C),
    out_shape=jax.ShapeDtypeStruct((NP, C), jnp.float32),
)


@jax.jit
def kernel(x, edge_index, W_feat, b_feat, lambdas, W_final, b_final):
    pad = jnp.full((EP - E,), N, jnp.int32)
    srcp = jnp.concatenate([edge_index[0], pad])
    dstp = jnp.concatenate([edge_index[1], pad])
    packed = (dstp << 16) | srcp
    pidx3 = packed.reshape(NW, NCHUNK, CH)
    dummy = jnp.full((MAXC, CH), (N << 16) | N, jnp.int32)
    pidxf = jnp.concatenate([packed.reshape(TOTC, CH), dummy])
    x_pad = jnp.zeros((NP, H), jnp.float32).at[:N].set(x)

    ones_d = jnp.ones((CH, H), jnp.float32)
    zeros_a = jnp.zeros((ROWS_PER_TILE, H), jnp.float32)

    degp = _deg_kernel(pidx3, ones_d, zeros_a)
    h0, g, dinv = _enc_call(x_pad, W_feat, b_feat.reshape(1, H), degp)

    p1 = _agg_kernel(g, pidxf, zeros_a)
    h1, g = _comb_call(h0, p1, dinv, lambdas[0].reshape(1, 1))

    p2 = _agg_kernel(g, pidxf, zeros_a)
    h2, g = _comb_call(h1, p2, dinv, lambdas[1].reshape(1, 1))

    p3 = _agg_kernel(g, pidxf, zeros_a)
    out = _final_call(h1, h2, p3, dinv, lambdas[2].reshape(1, 1),
                      W_final, b_final.reshape(1, C))
    return out[:N]
